# baseline (speedup 1.0000x reference)
import functools

import jax
import jax.numpy as jnp
from jax import lax
from jax.experimental import pallas as pl
from jax.experimental.pallas import tpu as pltpu
from jax.experimental.pallas import tpu_sc as plsc

N = 10000
E = 320000
D = 128
NC = 2
NS = 16
NW = NC * NS
CH = 128
N1 = 10240
NCHUNK = 80
NPAIR = NCHUNK // 2
EPT = NCHUNK * CH
EP = EPT * NW
RPS = N1 // NS

_mesh = plsc.VectorSubcoreMesh(
    core_axis_name="c", subcore_axis_name="s", num_cores=NC, num_subcores=NS
)


@functools.partial(
    pl.kernel,
    out_type=jax.ShapeDtypeStruct((NC, N1), jnp.float32),
    mesh=_mesh,
    scratch_types=[
        pltpu.VMEM((CH,), jnp.int32),
        pltpu.VMEM((CH,), jnp.float32),
        pltpu.VMEM((RPS,), jnp.float32),
        pltpu.VMEM_SHARED((N1,), jnp.float32),
    ],
)
def _deg_kernel(dst_hbm, out_hbm, di_v, ones_v, z_v, dacc_sh):
    c = lax.axis_index("c")
    s = lax.axis_index("s")
    w = s * NC + c

    def initbuf(j, carry):
        ones_v[pl.ds(j * 16, 16)] = jnp.ones((16,), jnp.float32)
        return carry

    lax.fori_loop(0, CH // 16, initbuf, 0)

    def zerobuf(j, carry):
        z_v[pl.ds(j * 16, 16)] = jnp.zeros((16,), jnp.float32)
        return carry

    lax.fori_loop(0, RPS // 16, zerobuf, 0)
    pltpu.sync_copy(z_v, dacc_sh.at[pl.ds(s * RPS, RPS)])
    plsc.subcore_barrier()

    def step(i, carry):
        pltpu.sync_copy(dst_hbm.at[pl.ds(w * EPT + i * CH, CH)], di_v)
        pltpu.sync_copy(ones_v, dacc_sh.at[di_v], add=True)
        return carry

    lax.fori_loop(0, NCHUNK, step, 0)
    plsc.subcore_barrier()
    pltpu.sync_copy(dacc_sh.at[pl.ds(s * RPS, RPS)],
                    out_hbm.at[c].at[pl.ds(s * RPS, RPS)])


@functools.partial(
    pl.kernel,
    out_type=jax.ShapeDtypeStruct((NC, N1, D), jnp.float32),
    mesh=_mesh,
    scratch_types=[
        pltpu.VMEM((CH,), jnp.int32),
        pltpu.VMEM((CH,), jnp.int32),
        pltpu.VMEM((CH,), jnp.int32),
        pltpu.VMEM((CH,), jnp.int32),
        pltpu.VMEM((CH, D), jnp.float32),
        pltpu.VMEM((CH, D), jnp.float32),
        pltpu.VMEM_SHARED((N1, D), jnp.float32),
        pltpu.SemaphoreType.DMA,
        pltpu.SemaphoreType.DMA,
    ],
)
def _hop_kernel(x_hbm, src_hbm, dst_hbm, zeros_hbm, out_hbm,
                si0_v, si1_v, di0_v, di1_v, rows0_v, rows1_v, acc_sh,
                semg0, semg1):
    c = lax.axis_index("c")
    s = lax.axis_index("s")
    w = s * NC + c

    pltpu.sync_copy(zeros_hbm.at[pl.ds(s * RPS, RPS)],
                    acc_sh.at[pl.ds(s * RPS, RPS)])
    plsc.subcore_barrier()

    def wait_rows(buf, sem):
        pltpu.make_async_copy(x_hbm.at[pl.ds(0, CH)], buf, sem).wait()

    def step(i, carry):
        off = w * EPT + i * CH
        pltpu.sync_copy(src_hbm.at[pl.ds(off, CH)], si0_v)
        pltpu.sync_copy(dst_hbm.at[pl.ds(off, CH)], di0_v)
        pltpu.async_copy(x_hbm.at[si0_v], rows0_v, semg0).wait()
        pltpu.sync_copy(rows0_v, acc_sh.at[di0_v], add=True)
        return carry

    lax.fori_loop(0, NCHUNK, step, 0)
    plsc.subcore_barrier()
    pltpu.sync_copy(acc_sh.at[pl.ds(s * RPS, RPS)],
                    out_hbm.at[c].at[pl.ds(s * RPS, RPS)])


def _norm_from(d0, d1):
    deg = d0 + d1
    return jnp.where(deg > 0, lax.rsqrt(jnp.maximum(deg, 1e-12)), 0.0)


def _prescale_body(f_ref, d0_ref, d1_ref, o_ref):
    o_ref[...] = f_ref[...] * _norm_from(d0_ref[...], d1_ref[...])


def _mid_body(p0_ref, p1_ref, d0_ref, d1_ref, o_ref):
    nrm = _norm_from(d0_ref[...], d1_ref[...])
    o_ref[...] = (p0_ref[...] + p1_ref[...]) * (nrm * nrm)


def _final_body(p0_ref, p1_ref, d0_ref, d1_ref, w_ref, b_ref, o_ref):
    h = (p0_ref[...] + p1_ref[...]) * _norm_from(d0_ref[...], d1_ref[...])
    o_ref[...] = (
        jnp.dot(h, w_ref[...], preferred_element_type=jnp.float32) + b_ref[...]
    )


_f32 = jnp.float32
_prescale = pl.pallas_call(
    _prescale_body, out_shape=jax.ShapeDtypeStruct((N1, D), _f32))
_mid = pl.pallas_call(
    _mid_body, out_shape=jax.ShapeDtypeStruct((N1, D), _f32))
_final = pl.pallas_call(
    _final_body, out_shape=jax.ShapeDtypeStruct((N1, D), _f32))


def kernel(feat, edge_index, W, b):
    src = edge_index[0].astype(jnp.int32)
    dst = edge_index[1].astype(jnp.int32)
    pad = EP - E
    srcp = jnp.concatenate([src, jnp.full((pad,), N, jnp.int32)])
    dstp = jnp.concatenate([dst, jnp.full((pad,), N, jnp.int32)])
    featp = jnp.concatenate(
        [feat.astype(_f32), jnp.zeros((N1 - N, D), _f32)])
    zeros2d = jnp.zeros((N1, D), _f32)

    degs = _deg_kernel(dstp)
    d0 = degs[0].reshape(N1, 1)
    d1 = degs[1].reshape(N1, 1)

    x0 = _prescale(featp, d0, d1)
    p = _hop_kernel(x0, srcp, dstp, zeros2d)
    x1 = _mid(p[0], p[1], d0, d1)
    q = _hop_kernel(x1, srcp, dstp, zeros2d)
    outp = _final(q[0], q[1], d0, d1, W.astype(_f32),
                  b.astype(_f32).reshape(1, D))
    return outp[:N]


# speedup vs baseline: 3.1507x; 3.1507x over previous
import functools

import jax
import jax.numpy as jnp
from jax import lax
from jax.experimental import pallas as pl
from jax.experimental.pallas import tpu as pltpu
from jax.experimental.pallas import tpu_sc as plsc

N = 10000
E = 320000
D = 128
NC = 2
NS = 16
NW = NC * NS
CH = 128
N1 = 10240
NCHUNK = 80
NPAIR = NCHUNK // 2
EPT = NCHUNK * CH
EP = EPT * NW
RPS = N1 // NS

_mesh = plsc.VectorSubcoreMesh(
    core_axis_name="c", subcore_axis_name="s", num_cores=NC, num_subcores=NS
)


@functools.partial(
    pl.kernel,
    out_type=jax.ShapeDtypeStruct((NC, N1), jnp.float32),
    mesh=_mesh,
    scratch_types=[
        pltpu.VMEM((CH,), jnp.int32),
        pltpu.VMEM((CH,), jnp.float32),
        pltpu.VMEM((RPS,), jnp.float32),
        pltpu.VMEM_SHARED((N1,), jnp.float32),
    ],
)
def _deg_kernel(dst_hbm, out_hbm, di_v, ones_v, z_v, dacc_sh):
    c = lax.axis_index("c")
    s = lax.axis_index("s")
    w = s * NC + c

    def initbuf(j, carry):
        ones_v[pl.ds(j * 16, 16)] = jnp.ones((16,), jnp.float32)
        return carry

    lax.fori_loop(0, CH // 16, initbuf, 0)

    def zerobuf(j, carry):
        z_v[pl.ds(j * 16, 16)] = jnp.zeros((16,), jnp.float32)
        return carry

    lax.fori_loop(0, RPS // 16, zerobuf, 0)
    pltpu.sync_copy(z_v, dacc_sh.at[pl.ds(s * RPS, RPS)])
    plsc.subcore_barrier()

    def step(i, carry):
        pltpu.sync_copy(dst_hbm.at[pl.ds(w * EPT + i * CH, CH)], di_v)
        pltpu.sync_copy(ones_v, dacc_sh.at[di_v], add=True)
        return carry

    lax.fori_loop(0, NCHUNK, step, 0)
    plsc.subcore_barrier()
    pltpu.sync_copy(dacc_sh.at[pl.ds(s * RPS, RPS)],
                    out_hbm.at[c].at[pl.ds(s * RPS, RPS)])


@functools.partial(
    pl.kernel,
    out_type=jax.ShapeDtypeStruct((NC, N1, D), jnp.float32),
    mesh=_mesh,
    scratch_types=[
        pltpu.VMEM((CH,), jnp.int32),
        pltpu.VMEM((CH,), jnp.int32),
        pltpu.VMEM((CH,), jnp.int32),
        pltpu.VMEM((CH,), jnp.int32),
        pltpu.VMEM((CH, D), jnp.float32),
        pltpu.VMEM((CH, D), jnp.float32),
        pltpu.VMEM_SHARED((N1, D), jnp.float32),
        pltpu.SemaphoreType.DMA,
        pltpu.SemaphoreType.DMA,
    ],
)
def _hop_kernel(x_hbm, src_hbm, dst_hbm, zeros_hbm, out_hbm,
                si0_v, si1_v, di0_v, di1_v, rows0_v, rows1_v, acc_sh,
                semg0, semg1):
    c = lax.axis_index("c")
    s = lax.axis_index("s")
    w = s * NC + c

    pltpu.sync_copy(zeros_hbm.at[pl.ds(s * RPS, RPS)],
                    acc_sh.at[pl.ds(s * RPS, RPS)])
    plsc.subcore_barrier()

    def wait_rows(buf, sem):
        pltpu.make_async_copy(x_hbm.at[pl.ds(0, CH)], buf, sem).wait()

    base = w * EPT
    pltpu.sync_copy(src_hbm.at[pl.ds(base, CH)], si0_v)
    pltpu.sync_copy(dst_hbm.at[pl.ds(base, CH)], di0_v)
    pltpu.async_copy(x_hbm.at[si0_v], rows0_v, semg0)

    def pair(i2, carry):
        off = base + 2 * i2 * CH
        pltpu.sync_copy(src_hbm.at[pl.ds(off + CH, CH)], si1_v)
        pltpu.sync_copy(dst_hbm.at[pl.ds(off + CH, CH)], di1_v)
        pltpu.async_copy(x_hbm.at[si1_v], rows1_v, semg1)
        wait_rows(rows0_v, semg0)
        pltpu.sync_copy(rows0_v, acc_sh.at[di0_v], add=True)
        pltpu.sync_copy(src_hbm.at[pl.ds(off + 2 * CH, CH)], si0_v)
        pltpu.sync_copy(dst_hbm.at[pl.ds(off + 2 * CH, CH)], di0_v)
        pltpu.async_copy(x_hbm.at[si0_v], rows0_v, semg0)
        wait_rows(rows1_v, semg1)
        pltpu.sync_copy(rows1_v, acc_sh.at[di1_v], add=True)
        return carry

    lax.fori_loop(0, NPAIR - 1, pair, 0)
    off = base + (NCHUNK - 1) * CH
    pltpu.sync_copy(src_hbm.at[pl.ds(off, CH)], si1_v)
    pltpu.sync_copy(dst_hbm.at[pl.ds(off, CH)], di1_v)
    pltpu.async_copy(x_hbm.at[si1_v], rows1_v, semg1)
    wait_rows(rows0_v, semg0)
    pltpu.sync_copy(rows0_v, acc_sh.at[di0_v], add=True)
    wait_rows(rows1_v, semg1)
    pltpu.sync_copy(rows1_v, acc_sh.at[di1_v], add=True)
    plsc.subcore_barrier()
    pltpu.sync_copy(acc_sh.at[pl.ds(s * RPS, RPS)],
                    out_hbm.at[c].at[pl.ds(s * RPS, RPS)])


def _norm_from(d0, d1):
    deg = d0 + d1
    return jnp.where(deg > 0, lax.rsqrt(jnp.maximum(deg, 1e-12)), 0.0)


def _prescale_body(f_ref, d0_ref, d1_ref, o_ref):
    o_ref[...] = f_ref[...] * _norm_from(d0_ref[...], d1_ref[...])


def _mid_body(p0_ref, p1_ref, d0_ref, d1_ref, o_ref):
    nrm = _norm_from(d0_ref[...], d1_ref[...])
    o_ref[...] = (p0_ref[...] + p1_ref[...]) * (nrm * nrm)


def _final_body(p0_ref, p1_ref, d0_ref, d1_ref, w_ref, b_ref, o_ref):
    h = (p0_ref[...] + p1_ref[...]) * _norm_from(d0_ref[...], d1_ref[...])
    o_ref[...] = (
        jnp.dot(h, w_ref[...], preferred_element_type=jnp.float32) + b_ref[...]
    )


_f32 = jnp.float32
_prescale = pl.pallas_call(
    _prescale_body, out_shape=jax.ShapeDtypeStruct((N1, D), _f32))
_mid = pl.pallas_call(
    _mid_body, out_shape=jax.ShapeDtypeStruct((N1, D), _f32))
_final = pl.pallas_call(
    _final_body, out_shape=jax.ShapeDtypeStruct((N1, D), _f32))


def kernel(feat, edge_index, W, b):
    src = edge_index[0].astype(jnp.int32)
    dst = edge_index[1].astype(jnp.int32)
    pad = EP - E
    padidx = N + (jnp.arange(pad, dtype=jnp.int32) % (N1 - N))
    srcp = jnp.concatenate([src, padidx])
    dstp = jnp.concatenate([dst, padidx])
    featp = jnp.concatenate(
        [feat.astype(_f32), jnp.zeros((N1 - N, D), _f32)])
    zeros2d = jnp.zeros((N1, D), _f32)

    degs = _deg_kernel(dstp)
    d0 = degs[0].reshape(N1, 1)
    d1 = degs[1].reshape(N1, 1)

    x0 = _prescale(featp, d0, d1)
    p = _hop_kernel(x0, srcp, dstp, zeros2d)
    x1 = _mid(p[0], p[1], d0, d1)
    q = _hop_kernel(x1, srcp, dstp, zeros2d)
    outp = _final(q[0], q[1], d0, d1, W.astype(_f32),
                  b.astype(_f32).reshape(1, D))
    return outp[:N]
